# baseline (device time: 54484 ns/iter reference)
import jax
import jax.numpy as jnp
from jax import lax
from jax.experimental import pallas as pl
from jax.experimental.pallas import tpu as pltpu

N_DEV = 4
B, SQ, SKV, DH = 2, 512, 512, 64
H_LOC = 8
D_LOC = H_LOC * DH
HALF = D_LOC // 2
D_MODEL = 768
BLK = 64

Q_B = (0, 0, 1, 1)
Q_HH = (0, 1, 0, 1)
SLOT_OF_TARGET = ((1, 2), (0, 2), (2, 0))


def kernel(x, Wq, K_ext, V_ext, Wo):
    def body(x_ref, wq_ref, k_ref, v_ref, wo_ref, out_ref,
             mine, mine8, mscale, inbox, inscale, ssem, rsem, ssem2, rsem2):
        my = lax.axis_index("i")
        left = lax.rem(my + N_DEV - 1, N_DEV)
        right = lax.rem(my + 1, N_DEV)
        opp = lax.rem(my + 2, N_DEV)

        xf = x_ref[...].reshape(B * SQ, D_MODEL).astype(jnp.bfloat16)
        q_half = {}

        def get_q(hh):
            if hh not in q_half:
                wq = wq_ref[:, pl.ds(my * D_LOC + hh * HALF, HALF)].astype(
                    jnp.bfloat16)
                qh = lax.dot_general(xf, wq, (((1,), (0,)), ((), ())),
                                     preferred_element_type=jnp.float32)
                q_half[hh] = (qh * 0.125).astype(jnp.bfloat16)
            return q_half[hh]

        qb = lax.broadcasted_iota(jnp.int32, (SQ, SKV), 0) // BLK
        kb = lax.broadcasted_iota(jnp.int32, (SQ, SKV), 1) // BLK
        mask = (qb == kb) | (kb == 0) | (lax.rem(qb + kb, 3) == 0)
        bias = jnp.where(mask, 0.0, -1e9).astype(jnp.bfloat16)

        aug_tail = jnp.concatenate(
            [jnp.ones((SKV, 1), jnp.bfloat16),
             jnp.zeros((SKV, 63), jnp.bfloat16)], axis=1)

        def attn_group(qtr):
            b, hh = Q_B[qtr], Q_HH[qtr]
            qh = get_q(hh)
            for hi in range(4):
                h = hh * 4 + hi
                q_bh = qh[b * SQ:(b + 1) * SQ, hi * DH:(hi + 1) * DH]
                k_bh = k_ref[b, :, h, :].astype(jnp.bfloat16)
                v_aug = jnp.concatenate(
                    [v_ref[b, :, h, :].astype(jnp.bfloat16), aug_tail],
                    axis=1)
                s = lax.dot_general(q_bh, k_bh, (((1,), (1,)), ((), ())),
                                    preferred_element_type=jnp.float32)
                w = jnp.exp(s.astype(jnp.bfloat16) + bias)
                ctx_aug = lax.dot_general(w, v_aug, (((1,), (0,)), ((), ())),
                                          preferred_element_type=jnp.float32)
                ctx = ctx_aug[:, :DH] * (1.0 / ctx_aug[:, DH:DH + 1])
                mine[qtr, :, hi * DH:(hi + 1) * DH] = ctx.astype(jnp.bfloat16)

        rdmas = {}

        def send_quarter(qtr):
            c = mine[qtr].astype(jnp.float32)
            amax = jnp.max(jnp.abs(c), axis=1, keepdims=True) + 1e-6
            mscale[qtr] = (amax * (1.0 / 127.0)).astype(jnp.float32)
            qv = jnp.round(c * (127.0 / amax))
            mine8[qtr] = jnp.clip(qv, -127.0, 127.0).astype(jnp.int8)
            for ti, (dev, slot) in enumerate(
                    ((left, 1), (right, 0), (opp, 2))):
                rd = pltpu.make_async_remote_copy(
                    src_ref=mine8.at[qtr], dst_ref=inbox.at[slot, qtr],
                    send_sem=ssem.at[qtr, ti], recv_sem=rsem.at[slot, qtr],
                    device_id=(dev,), device_id_type=pl.DeviceIdType.MESH)
                rd.start()
                rdmas[(qtr, ti)] = rd
                rs = pltpu.make_async_remote_copy(
                    src_ref=mscale.at[qtr], dst_ref=inscale.at[slot, qtr],
                    send_sem=ssem2.at[qtr, ti], recv_sem=rsem2.at[slot, qtr],
                    device_id=(dev,), device_id_type=pl.DeviceIdType.MESH)
                rs.start()
                rdmas[(qtr, ti, 's')] = rs

        def wo_q(qtr, dev_idx):
            off = Q_HH[qtr] * HALF
            return wo_ref[pl.ds(dev_idx * D_LOC + off, HALF), :].astype(
                jnp.bfloat16)

        def qdot(src_ref, qtr, dev_idx):
            return lax.dot_general(src_ref, wo_q(qtr, dev_idx),
                                   (((1,), (0,)), ((), ())),
                                   preferred_element_type=jnp.float32)

        for qtr in range(4):
            attn_group(qtr)
            if qtr == 0:
                barrier = pltpu.get_barrier_semaphore()
                for nbr in (left, right, opp):
                    pl.semaphore_signal(barrier, inc=1, device_id=(nbr,),
                                        device_id_type=pl.DeviceIdType.MESH)
                pl.semaphore_wait(barrier, 3)
            send_quarter(qtr)

        acc0 = qdot(mine[0], 0, my) + qdot(mine[1], 1, my)
        acc1 = qdot(mine[2], 2, my) + qdot(mine[3], 3, my)

        accs = [acc0, acc1]
        src_dev = (left, right, opp)
        for qtr in range(4):
            for slot in range(3):
                recv = pltpu.make_async_remote_copy(
                    src_ref=mine8.at[qtr], dst_ref=inbox.at[slot, qtr],
                    send_sem=ssem.at[qtr, slot], recv_sem=rsem.at[slot, qtr],
                    device_id=(src_dev[slot],),
                    device_id_type=pl.DeviceIdType.MESH)
                recv.wait_recv()
                recv_s = pltpu.make_async_remote_copy(
                    src_ref=mscale.at[qtr], dst_ref=inscale.at[slot, qtr],
                    send_sem=ssem2.at[qtr, slot],
                    recv_sem=rsem2.at[slot, qtr],
                    device_id=(src_dev[slot],),
                    device_id_type=pl.DeviceIdType.MESH)
                recv_s.wait_recv()
                deq = (inbox[slot, qtr].astype(jnp.bfloat16)
                       * inscale[slot, qtr].astype(jnp.bfloat16))
                b = Q_B[qtr]
                accs[b] = accs[b] + qdot(deq, qtr, src_dev[slot])

        for key in rdmas:
            rdmas[key].wait_send()

        out_ref[0] = accs[0]
        out_ref[1] = accs[1]

    return pl.pallas_call(
        body,
        out_shape=jax.ShapeDtypeStruct((B, SQ, D_MODEL), jnp.float32),
        in_specs=[pl.BlockSpec(memory_space=pltpu.VMEM)] * 5,
        out_specs=pl.BlockSpec(memory_space=pltpu.VMEM),
        scratch_shapes=[
            pltpu.VMEM((4, SQ, HALF), jnp.bfloat16),
            pltpu.VMEM((4, SQ, HALF), jnp.int8),
            pltpu.VMEM((4, SQ, 1), jnp.float32),
            pltpu.VMEM((3, 4, SQ, HALF), jnp.int8),
            pltpu.VMEM((3, 4, SQ, 1), jnp.float32),
            pltpu.SemaphoreType.DMA((4, 3)),
            pltpu.SemaphoreType.DMA((3, 4)),
            pltpu.SemaphoreType.DMA((4, 3)),
            pltpu.SemaphoreType.DMA((3, 4)),
        ],
        compiler_params=pltpu.CompilerParams(collective_id=0),
    )(x, Wq, K_ext, V_ext, Wo)
